# baseline (speedup 1.0000x reference)
import functools

import jax
import jax.numpy as jnp
from jax import lax
from jax.experimental import pallas as pl
from jax.experimental.pallas import tpu as pltpu
from jax.experimental.pallas import tpu_sc as plsc

NC = 2
NS = 16
NW = NC * NS
N = 10000
NPAD = 10240
D = 128
H = D // NC
E = 320000
EP = E // NS
K = 80
CHUNKS = EP // K
SUPC = 50
SUP = CHUNKS // SUPC
NBUF = 5
UNROLL = 10
INNER = SUPC // UNROLL
SLAG = 2
GLEAD = NBUF - SLAG
ROWS_PER_TILE = NPAD // NS


def _sc_segment_sum(x0, x1, src3d, dst3d, *, with_deg):
    mesh = plsc.VectorSubcoreMesh(core_axis_name="c", subcore_axis_name="s")

    out_type = [
        jax.ShapeDtypeStruct((NPAD, H), jnp.float32),
        jax.ShapeDtypeStruct((NPAD, H), jnp.float32),
    ]
    if with_deg:
        out_type.append(jax.ShapeDtypeStruct((NW * NPAD,), jnp.float32))

    scratch = [
        pltpu.VMEM_SHARED((NPAD, H), jnp.float32),
        pltpu.VMEM((SUPC, K), jnp.int32),
        pltpu.VMEM((SUPC, K), jnp.int32),
    ] + [pltpu.VMEM((K, H), jnp.float32) for _ in range(NBUF)] + [
        pltpu.VMEM((NPAD,), jnp.float32),
    ] + [pltpu.SemaphoreType.DMA for _ in range(2 * NBUF)]

    def body(x0_hbm, x1_hbm, src_hbm, dst_hbm, agg0_hbm, agg1_hbm, *rest):
        if with_deg:
            degp_hbm = rest[0]
            scr = rest[1:]
        else:
            scr = rest
        agg_sp = scr[0]
        src_v = scr[1]
        dst_v = scr[2]
        bufs = scr[3:3 + NBUF]
        deg_l = scr[3 + NBUF]
        gsem = scr[4 + NBUF:4 + 2 * NBUF]
        ssem = scr[4 + 2 * NBUF:4 + 3 * NBUF]
        rows_v = bufs[0]

        c = lax.axis_index("c")
        s = lax.axis_index("s")
        w = s * NC + c
        zeros16 = jnp.zeros((16,), jnp.float32)
        ones16 = jnp.ones((16,), jnp.float32)

        def zero_rb(i, carry):
            r = i // (H // 16)
            q = lax.rem(i, H // 16)
            rows_v[r, pl.ds(q * 16, 16)] = zeros16
            return carry
        lax.fori_loop(0, K * (H // 16), zero_rb, 0)
        for b in range(ROWS_PER_TILE // K):
            pltpu.sync_copy(rows_v, agg_sp.at[pl.ds(s * ROWS_PER_TILE + b * K, K)])

        if with_deg:
            def zero_deg(i, carry):
                deg_l[pl.ds(i * 16, 16)] = zeros16
                return carry
            lax.fori_loop(0, NPAD // 16, zero_deg, 0)

        plsc.subcore_barrier()

        def main_loop(xh):
            def super_chunk(g, carry):
                pltpu.sync_copy(src_hbm.at[s, pl.ds(g * SUPC, SUPC)], src_v)
                pltpu.sync_copy(dst_hbm.at[s, pl.ds(g * SUPC, SUPC)], dst_v)
                if with_deg:
                    def deg_iter(j, carry2):
                        for q in range(K // 16):
                            idx = dst_v[j, pl.ds(q * 16, 16)]
                            plsc.addupdate_scatter(deg_l, [idx], ones16)
                        return carry2
                    lax.fori_loop(c * (SUPC // NC), (c + 1) * (SUPC // NC),
                                  deg_iter, 0)
                for r0 in range(GLEAD):
                    pltpu.async_copy(xh.at[src_v.at[r0]], bufs[r0], gsem[r0])

                def inner(p, carry2):
                    for b10 in range(UNROLL):
                        r = p * UNROLL + b10
                        B = b10 % NBUF
                        BF = (b10 + NBUF - SLAG) % NBUF
                        pltpu.make_async_copy(
                            xh.at[src_v.at[r]], bufs[B], gsem[B]).wait()
                        pltpu.async_copy(
                            bufs[B], agg_sp.at[dst_v.at[r]], ssem[B], add=True)
                        if b10 < SLAG:
                            @pl.when(p > 0)
                            def _():
                                pltpu.make_async_copy(
                                    bufs[BF], agg_sp.at[dst_v.at[r - SLAG]],
                                    ssem[BF]).wait()
                        else:
                            pltpu.make_async_copy(
                                bufs[BF], agg_sp.at[dst_v.at[r - SLAG]],
                                ssem[BF]).wait()
                        if b10 < UNROLL - GLEAD:
                            pltpu.async_copy(
                                xh.at[src_v.at[r + GLEAD]], bufs[BF],
                                gsem[BF])
                        else:
                            @pl.when(p < INNER - 1)
                            def _():
                                pltpu.async_copy(
                                    xh.at[src_v.at[r + GLEAD]], bufs[BF],
                                    gsem[BF])
                    return carry2
                lax.fori_loop(0, INNER, inner, 0)
                for t in range(SLAG):
                    rt = SUPC - SLAG + t
                    BT = rt % NBUF
                    pltpu.make_async_copy(
                        bufs[BT], agg_sp.at[dst_v.at[rt]], ssem[BT]).wait()
                return carry
            lax.fori_loop(0, SUP, super_chunk, 0)

        @pl.when(c == 0)
        def _():
            main_loop(x0_hbm)
        @pl.when(c == 1)
        def _():
            main_loop(x1_hbm)

        if with_deg:
            pltpu.sync_copy(deg_l, degp_hbm.at[pl.ds(w * NPAD, NPAD)])

        plsc.subcore_barrier()

        for ci, agg_hbm in ((0, agg0_hbm), (1, agg1_hbm)):
            @pl.when(c == ci)
            def _(agg_hbm=agg_hbm):
                pltpu.sync_copy(
                    agg_sp.at[pl.ds(s * ROWS_PER_TILE, ROWS_PER_TILE)],
                    agg_hbm.at[pl.ds(s * ROWS_PER_TILE, ROWS_PER_TILE)],
                )

    f = pl.kernel(body, out_type=tuple(out_type), mesh=mesh,
                  scratch_types=scratch,
                  compiler_params=pltpu.CompilerParams(
                      needs_layout_passes=False,
                      use_tc_tiling_on_sc=False))
    return f(x0, x1, src3d, dst3d)


def _tc_recip_deg(deg_parts):
    def body(degp_ref, recip_ref):
        deg = jnp.sum(degp_ref[...], axis=0)
        recip_ref[...] = (1.0 / jnp.maximum(deg, 1.0))[:, None]

    return pl.pallas_call(
        body,
        out_shape=jax.ShapeDtypeStruct((NPAD, 1), jnp.float32),
    )(deg_parts)


def _tc_right(x0, x1, W_r, b):
    RB = 1000

    def body(x0_ref, x1_ref, wr_ref, b_ref, z_ref):
        xx = jnp.concatenate([x0_ref[...], x1_ref[...]], axis=1)
        z_ref[...] = (
            jnp.dot(xx, wr_ref[...], preferred_element_type=jnp.float32)
            + b_ref[...]
        )

    return pl.pallas_call(
        body,
        grid=(N // RB,),
        in_specs=[
            pl.BlockSpec((RB, H), lambda i: (i, 0)),
            pl.BlockSpec((RB, H), lambda i: (i, 0)),
            pl.BlockSpec((D, D), lambda i: (0, 0)),
            pl.BlockSpec((1, D), lambda i: (0, 0)),
        ],
        out_specs=pl.BlockSpec((RB, D), lambda i: (i, 0)),
        out_shape=jax.ShapeDtypeStruct((N, D), jnp.float32),
    )(x0, x1, W_r, b)


def _tc_left(agg0, agg1, recip, z, W_l, relu, split_out):
    RB = 1000

    def body(agg0_ref, agg1_ref, recip_ref, z_ref, wl_ref, *out_refs):
        agg = jnp.concatenate([agg0_ref[...], agg1_ref[...]], axis=1)
        mean = agg * recip_ref[...]
        out = (
            jnp.dot(mean, wl_ref[...], preferred_element_type=jnp.float32)
            + z_ref[...]
        )
        if relu:
            out = jnp.maximum(out, 0.0)
        if split_out:
            out_refs[0][...] = out[:, :H]
            out_refs[1][...] = out[:, H:]
        else:
            out_refs[0][...] = out

    if split_out:
        out_specs = [
            pl.BlockSpec((RB, H), lambda i: (i, 0)),
            pl.BlockSpec((RB, H), lambda i: (i, 0)),
        ]
        out_shape = [
            jax.ShapeDtypeStruct((N, H), jnp.float32),
            jax.ShapeDtypeStruct((N, H), jnp.float32),
        ]
    else:
        out_specs = pl.BlockSpec((RB, D), lambda i: (i, 0))
        out_shape = jax.ShapeDtypeStruct((N, D), jnp.float32)

    return pl.pallas_call(
        body,
        grid=(N // RB,),
        in_specs=[
            pl.BlockSpec((RB, H), lambda i: (i, 0)),
            pl.BlockSpec((RB, H), lambda i: (i, 0)),
            pl.BlockSpec((RB, 1), lambda i: (i, 0)),
            pl.BlockSpec((RB, D), lambda i: (i, 0)),
            pl.BlockSpec((D, D), lambda i: (0, 0)),
        ],
        out_specs=out_specs,
        out_shape=out_shape,
    )(agg0, agg1, recip, z, W_l)


@jax.jit
def kernel(x, edge_index, W1_l, b1_l, W1_r, W2_l, b2_l, W2_r):
    src = edge_index[0].astype(jnp.int32)
    dst = edge_index[1].astype(jnp.int32)
    src3d = src.reshape(NS, CHUNKS, K)
    dst3d = dst.reshape(NS, CHUNKS, K)
    x0 = x[:, :H]
    x1 = x[:, H:]

    z1 = _tc_right(x0, x1, W1_r, b1_l.reshape(1, D))
    agg1_0, agg1_1, deg_parts = _sc_segment_sum(
        x0, x1, src3d, dst3d, with_deg=True)
    recip = _tc_recip_deg(deg_parts.reshape(NW, NPAD))
    h0, h1 = _tc_left(agg1_0, agg1_1, recip, z1, W1_l, relu=True,
                      split_out=True)
    z2 = _tc_right(h0, h1, W2_r, b2_l.reshape(1, D))
    agg2_0, agg2_1 = _sc_segment_sum(h0, h1, src3d, dst3d, with_deg=False)
    return _tc_left(agg2_0, agg2_1, recip, z2, W2_l, relu=False,
                    split_out=False)


# speedup vs baseline: 1.0581x; 1.0581x over previous
import functools

import jax
import jax.numpy as jnp
from jax import lax
from jax.experimental import pallas as pl
from jax.experimental.pallas import tpu as pltpu
from jax.experimental.pallas import tpu_sc as plsc

NC = 2
NS = 16
NW = NC * NS
N = 10000
NPAD = 10240
D = 128
H = D // NC
E = 320000
EP = E // NS
K = 80
CHUNKS = EP // K
SUPC = 50
SUP = CHUNKS // SUPC
NBUF = 5
UNROLL = 10
INNER = SUPC // UNROLL
SLAG = 1
GLEAD = NBUF - SLAG
ROWS_PER_TILE = NPAD // NS


def _sc_segment_sum(x0, x1, src3d, dst3d, *, with_deg):
    mesh = plsc.VectorSubcoreMesh(core_axis_name="c", subcore_axis_name="s")

    out_type = [
        jax.ShapeDtypeStruct((NPAD, H), jnp.float32),
        jax.ShapeDtypeStruct((NPAD, H), jnp.float32),
    ]
    if with_deg:
        out_type.append(jax.ShapeDtypeStruct((NW * NPAD,), jnp.float32))

    scratch = [
        pltpu.VMEM_SHARED((NPAD, H), jnp.float32),
        pltpu.VMEM((SUPC, K), jnp.int32),
        pltpu.VMEM((SUPC, K), jnp.int32),
    ] + [pltpu.VMEM((K, H), jnp.float32) for _ in range(NBUF)] + [
        pltpu.VMEM((NPAD,), jnp.float32),
    ] + [pltpu.SemaphoreType.DMA for _ in range(2 * NBUF)]

    def body(x0_hbm, x1_hbm, src_hbm, dst_hbm, agg0_hbm, agg1_hbm, *rest):
        if with_deg:
            degp_hbm = rest[0]
            scr = rest[1:]
        else:
            scr = rest
        agg_sp = scr[0]
        src_v = scr[1]
        dst_v = scr[2]
        bufs = scr[3:3 + NBUF]
        deg_l = scr[3 + NBUF]
        gsem = scr[4 + NBUF:4 + 2 * NBUF]
        ssem = scr[4 + 2 * NBUF:4 + 3 * NBUF]
        rows_v = bufs[0]

        c = lax.axis_index("c")
        s = lax.axis_index("s")
        w = s * NC + c
        zeros16 = jnp.zeros((16,), jnp.float32)
        ones16 = jnp.ones((16,), jnp.float32)

        def zero_rb(i, carry):
            r = i // (H // 16)
            q = lax.rem(i, H // 16)
            rows_v[r, pl.ds(q * 16, 16)] = zeros16
            return carry
        lax.fori_loop(0, K * (H // 16), zero_rb, 0)
        for b in range(ROWS_PER_TILE // K):
            pltpu.sync_copy(rows_v, agg_sp.at[pl.ds(s * ROWS_PER_TILE + b * K, K)])

        if with_deg:
            def zero_deg(i, carry):
                deg_l[pl.ds(i * 16, 16)] = zeros16
                return carry
            lax.fori_loop(0, NPAD // 16, zero_deg, 0)

        plsc.subcore_barrier()

        def main_loop(xh):
            def super_chunk(g, carry):
                pltpu.sync_copy(src_hbm.at[s, pl.ds(g * SUPC, SUPC)], src_v)
                pltpu.sync_copy(dst_hbm.at[s, pl.ds(g * SUPC, SUPC)], dst_v)
                if with_deg:
                    def deg_iter(j, carry2):
                        for q in range(K // 16):
                            idx = dst_v[j, pl.ds(q * 16, 16)]
                            plsc.addupdate_scatter(deg_l, [idx], ones16)
                        return carry2
                    lax.fori_loop(c * (SUPC // NC), (c + 1) * (SUPC // NC),
                                  deg_iter, 0)
                for r0 in range(GLEAD):
                    pltpu.async_copy(xh.at[src_v.at[r0]], bufs[r0], gsem[r0])

                def inner(p, carry2):
                    for b10 in range(UNROLL):
                        r = p * UNROLL + b10
                        B = b10 % NBUF
                        BF = (b10 + NBUF - SLAG) % NBUF
                        pltpu.make_async_copy(
                            xh.at[src_v.at[r]], bufs[B], gsem[B]).wait()
                        pltpu.async_copy(
                            bufs[B], agg_sp.at[dst_v.at[r]], ssem[B], add=True)
                        if b10 < SLAG:
                            @pl.when(p > 0)
                            def _():
                                pltpu.make_async_copy(
                                    bufs[BF], agg_sp.at[dst_v.at[r - SLAG]],
                                    ssem[BF]).wait()
                        else:
                            pltpu.make_async_copy(
                                bufs[BF], agg_sp.at[dst_v.at[r - SLAG]],
                                ssem[BF]).wait()
                        if b10 < UNROLL - GLEAD:
                            pltpu.async_copy(
                                xh.at[src_v.at[r + GLEAD]], bufs[BF],
                                gsem[BF])
                        else:
                            @pl.when(p < INNER - 1)
                            def _():
                                pltpu.async_copy(
                                    xh.at[src_v.at[r + GLEAD]], bufs[BF],
                                    gsem[BF])
                    return carry2
                lax.fori_loop(0, INNER, inner, 0)
                for t in range(SLAG):
                    rt = SUPC - SLAG + t
                    BT = rt % NBUF
                    pltpu.make_async_copy(
                        bufs[BT], agg_sp.at[dst_v.at[rt]], ssem[BT]).wait()
                return carry
            lax.fori_loop(0, SUP, super_chunk, 0)

        @pl.when(c == 0)
        def _():
            main_loop(x0_hbm)
        @pl.when(c == 1)
        def _():
            main_loop(x1_hbm)

        if with_deg:
            pltpu.sync_copy(deg_l, degp_hbm.at[pl.ds(w * NPAD, NPAD)])

        plsc.subcore_barrier()

        for ci, agg_hbm in ((0, agg0_hbm), (1, agg1_hbm)):
            @pl.when(c == ci)
            def _(agg_hbm=agg_hbm):
                pltpu.sync_copy(
                    agg_sp.at[pl.ds(s * ROWS_PER_TILE, ROWS_PER_TILE)],
                    agg_hbm.at[pl.ds(s * ROWS_PER_TILE, ROWS_PER_TILE)],
                )

    f = pl.kernel(body, out_type=tuple(out_type), mesh=mesh,
                  scratch_types=scratch,
                  compiler_params=pltpu.CompilerParams(
                      needs_layout_passes=False,
                      use_tc_tiling_on_sc=False))
    return f(x0, x1, src3d, dst3d)


def _tc_recip_deg(deg_parts):
    def body(degp_ref, recip_ref):
        deg = jnp.sum(degp_ref[...], axis=0)
        recip_ref[...] = (1.0 / jnp.maximum(deg, 1.0))[:, None]

    return pl.pallas_call(
        body,
        out_shape=jax.ShapeDtypeStruct((NPAD, 1), jnp.float32),
    )(deg_parts)


def _tc_right(x0, x1, W_r, b):
    RB = 1000

    def body(x0_ref, x1_ref, wr_ref, b_ref, z_ref):
        xx = jnp.concatenate([x0_ref[...], x1_ref[...]], axis=1)
        z_ref[...] = (
            jnp.dot(xx, wr_ref[...], preferred_element_type=jnp.float32)
            + b_ref[...]
        )

    return pl.pallas_call(
        body,
        grid=(N // RB,),
        in_specs=[
            pl.BlockSpec((RB, H), lambda i: (i, 0)),
            pl.BlockSpec((RB, H), lambda i: (i, 0)),
            pl.BlockSpec((D, D), lambda i: (0, 0)),
            pl.BlockSpec((1, D), lambda i: (0, 0)),
        ],
        out_specs=pl.BlockSpec((RB, D), lambda i: (i, 0)),
        out_shape=jax.ShapeDtypeStruct((N, D), jnp.float32),
    )(x0, x1, W_r, b)


def _tc_left(agg0, agg1, recip, z, W_l, relu, split_out):
    RB = 1000

    def body(agg0_ref, agg1_ref, recip_ref, z_ref, wl_ref, *out_refs):
        agg = jnp.concatenate([agg0_ref[...], agg1_ref[...]], axis=1)
        mean = agg * recip_ref[...]
        out = (
            jnp.dot(mean, wl_ref[...], preferred_element_type=jnp.float32)
            + z_ref[...]
        )
        if relu:
            out = jnp.maximum(out, 0.0)
        if split_out:
            out_refs[0][...] = out[:, :H]
            out_refs[1][...] = out[:, H:]
        else:
            out_refs[0][...] = out

    if split_out:
        out_specs = [
            pl.BlockSpec((RB, H), lambda i: (i, 0)),
            pl.BlockSpec((RB, H), lambda i: (i, 0)),
        ]
        out_shape = [
            jax.ShapeDtypeStruct((N, H), jnp.float32),
            jax.ShapeDtypeStruct((N, H), jnp.float32),
        ]
    else:
        out_specs = pl.BlockSpec((RB, D), lambda i: (i, 0))
        out_shape = jax.ShapeDtypeStruct((N, D), jnp.float32)

    return pl.pallas_call(
        body,
        grid=(N // RB,),
        in_specs=[
            pl.BlockSpec((RB, H), lambda i: (i, 0)),
            pl.BlockSpec((RB, H), lambda i: (i, 0)),
            pl.BlockSpec((RB, 1), lambda i: (i, 0)),
            pl.BlockSpec((RB, D), lambda i: (i, 0)),
            pl.BlockSpec((D, D), lambda i: (0, 0)),
        ],
        out_specs=out_specs,
        out_shape=out_shape,
    )(agg0, agg1, recip, z, W_l)


@jax.jit
def kernel(x, edge_index, W1_l, b1_l, W1_r, W2_l, b2_l, W2_r):
    src = edge_index[0].astype(jnp.int32)
    dst = edge_index[1].astype(jnp.int32)
    src3d = src.reshape(NS, CHUNKS, K)
    dst3d = dst.reshape(NS, CHUNKS, K)
    x0 = x[:, :H]
    x1 = x[:, H:]

    z1 = _tc_right(x0, x1, W1_r, b1_l.reshape(1, D))
    agg1_0, agg1_1, deg_parts = _sc_segment_sum(
        x0, x1, src3d, dst3d, with_deg=True)
    recip = _tc_recip_deg(deg_parts.reshape(NW, NPAD))
    h0, h1 = _tc_left(agg1_0, agg1_1, recip, z1, W1_l, relu=True,
                      split_out=True)
    z2 = _tc_right(h0, h1, W2_r, b2_l.reshape(1, D))
    agg2_0, agg2_1 = _sc_segment_sum(h0, h1, src3d, dst3d, with_deg=False)
    return _tc_left(agg2_0, agg2_1, recip, z2, W2_l, relu=False,
                    split_out=False)


# speedup vs baseline: 1.1169x; 1.0555x over previous
import functools

import jax
import jax.numpy as jnp
from jax import lax
from jax.experimental import pallas as pl
from jax.experimental.pallas import tpu as pltpu
from jax.experimental.pallas import tpu_sc as plsc

NC = 2
NS = 16
NW = NC * NS
N = 10000
NPAD = 10240
D = 128
H = D // NC
E = 320000
EP = E // NS
K = 80
CHUNKS = EP // K
SUPC = 250
SUP = CHUNKS // SUPC
NBUF = 5
UNROLL = 10
INNER = SUPC // UNROLL
SLAG = 1
GLEAD = NBUF - SLAG
ROWS_PER_TILE = NPAD // NS


def _sc_segment_sum(x0, x1, src3d, dst3d, *, with_deg):
    mesh = plsc.VectorSubcoreMesh(core_axis_name="c", subcore_axis_name="s")

    out_type = [
        jax.ShapeDtypeStruct((NPAD, H), jnp.float32),
        jax.ShapeDtypeStruct((NPAD, H), jnp.float32),
    ]
    if with_deg:
        out_type.append(jax.ShapeDtypeStruct((NW * NPAD,), jnp.float32))

    scratch = [
        pltpu.VMEM_SHARED((NPAD, H), jnp.float32),
        pltpu.VMEM((SUPC, K), jnp.int32),
        pltpu.VMEM((SUPC, K), jnp.int32),
    ] + [pltpu.VMEM((K, H), jnp.float32) for _ in range(NBUF)] + [
        pltpu.VMEM((NPAD,), jnp.float32),
    ] + [pltpu.SemaphoreType.DMA for _ in range(2 * NBUF)]

    def body(x0_hbm, x1_hbm, src_hbm, dst_hbm, agg0_hbm, agg1_hbm, *rest):
        if with_deg:
            degp_hbm = rest[0]
            scr = rest[1:]
        else:
            scr = rest
        agg_sp = scr[0]
        src_v = scr[1]
        dst_v = scr[2]
        bufs = scr[3:3 + NBUF]
        deg_l = scr[3 + NBUF]
        gsem = scr[4 + NBUF:4 + 2 * NBUF]
        ssem = scr[4 + 2 * NBUF:4 + 3 * NBUF]
        rows_v = bufs[0]

        c = lax.axis_index("c")
        s = lax.axis_index("s")
        w = s * NC + c
        zeros16 = jnp.zeros((16,), jnp.float32)
        ones16 = jnp.ones((16,), jnp.float32)

        def zero_rb(i, carry):
            r = i // (H // 16)
            q = lax.rem(i, H // 16)
            rows_v[r, pl.ds(q * 16, 16)] = zeros16
            return carry
        lax.fori_loop(0, K * (H // 16), zero_rb, 0)
        for b in range(ROWS_PER_TILE // K):
            pltpu.sync_copy(rows_v, agg_sp.at[pl.ds(s * ROWS_PER_TILE + b * K, K)])

        if with_deg:
            def zero_deg(i, carry):
                deg_l[pl.ds(i * 16, 16)] = zeros16
                return carry
            lax.fori_loop(0, NPAD // 16, zero_deg, 0)

        plsc.subcore_barrier()

        def main_loop(xh):
            def super_chunk(g, carry):
                pltpu.sync_copy(src_hbm.at[s, pl.ds(g * SUPC, SUPC)], src_v)
                pltpu.sync_copy(dst_hbm.at[s, pl.ds(g * SUPC, SUPC)], dst_v)
                if with_deg:
                    def deg_iter(j, carry2):
                        for q in range(K // 16):
                            idx = dst_v[j, pl.ds(q * 16, 16)]
                            plsc.addupdate_scatter(deg_l, [idx], ones16)
                        return carry2
                    lax.fori_loop(c * (SUPC // NC), (c + 1) * (SUPC // NC),
                                  deg_iter, 0)
                for r0 in range(GLEAD):
                    pltpu.async_copy(xh.at[src_v.at[r0]], bufs[r0], gsem[r0])

                def inner(p, carry2):
                    for b10 in range(UNROLL):
                        r = p * UNROLL + b10
                        B = b10 % NBUF
                        BF = (b10 + NBUF - SLAG) % NBUF
                        pltpu.make_async_copy(
                            xh.at[src_v.at[r]], bufs[B], gsem[B]).wait()
                        pltpu.async_copy(
                            bufs[B], agg_sp.at[dst_v.at[r]], ssem[B], add=True)
                        if b10 < SLAG:
                            @pl.when(p > 0)
                            def _():
                                pltpu.make_async_copy(
                                    bufs[BF], agg_sp.at[dst_v.at[r - SLAG]],
                                    ssem[BF]).wait()
                        else:
                            pltpu.make_async_copy(
                                bufs[BF], agg_sp.at[dst_v.at[r - SLAG]],
                                ssem[BF]).wait()
                        if b10 < UNROLL - GLEAD:
                            pltpu.async_copy(
                                xh.at[src_v.at[r + GLEAD]], bufs[BF],
                                gsem[BF])
                        else:
                            @pl.when(p < INNER - 1)
                            def _():
                                pltpu.async_copy(
                                    xh.at[src_v.at[r + GLEAD]], bufs[BF],
                                    gsem[BF])
                    return carry2
                lax.fori_loop(0, INNER, inner, 0)
                for t in range(SLAG):
                    rt = SUPC - SLAG + t
                    BT = rt % NBUF
                    pltpu.make_async_copy(
                        bufs[BT], agg_sp.at[dst_v.at[rt]], ssem[BT]).wait()
                return carry
            lax.fori_loop(0, SUP, super_chunk, 0)

        @pl.when(c == 0)
        def _():
            main_loop(x0_hbm)
        @pl.when(c == 1)
        def _():
            main_loop(x1_hbm)

        if with_deg:
            pltpu.sync_copy(deg_l, degp_hbm.at[pl.ds(w * NPAD, NPAD)])

        plsc.subcore_barrier()

        for ci, agg_hbm in ((0, agg0_hbm), (1, agg1_hbm)):
            @pl.when(c == ci)
            def _(agg_hbm=agg_hbm):
                pltpu.sync_copy(
                    agg_sp.at[pl.ds(s * ROWS_PER_TILE, ROWS_PER_TILE)],
                    agg_hbm.at[pl.ds(s * ROWS_PER_TILE, ROWS_PER_TILE)],
                )

    f = pl.kernel(body, out_type=tuple(out_type), mesh=mesh,
                  scratch_types=scratch,
                  compiler_params=pltpu.CompilerParams(
                      needs_layout_passes=False,
                      use_tc_tiling_on_sc=False))
    return f(x0, x1, src3d, dst3d)


def _tc_recip_deg(deg_parts):
    def body(degp_ref, recip_ref):
        deg = jnp.sum(degp_ref[...], axis=0)
        recip_ref[...] = (1.0 / jnp.maximum(deg, 1.0))[:, None]

    return pl.pallas_call(
        body,
        out_shape=jax.ShapeDtypeStruct((NPAD, 1), jnp.float32),
    )(deg_parts)


def _tc_right(x0, x1, W_r, b):
    RB = 1000

    def body(x0_ref, x1_ref, wr_ref, b_ref, z_ref):
        xx = jnp.concatenate([x0_ref[...], x1_ref[...]], axis=1)
        z_ref[...] = (
            jnp.dot(xx, wr_ref[...], preferred_element_type=jnp.float32)
            + b_ref[...]
        )

    return pl.pallas_call(
        body,
        grid=(N // RB,),
        in_specs=[
            pl.BlockSpec((RB, H), lambda i: (i, 0)),
            pl.BlockSpec((RB, H), lambda i: (i, 0)),
            pl.BlockSpec((D, D), lambda i: (0, 0)),
            pl.BlockSpec((1, D), lambda i: (0, 0)),
        ],
        out_specs=pl.BlockSpec((RB, D), lambda i: (i, 0)),
        out_shape=jax.ShapeDtypeStruct((N, D), jnp.float32),
    )(x0, x1, W_r, b)


def _tc_left(agg0, agg1, recip, z, W_l, relu, split_out):
    RB = 1000

    def body(agg0_ref, agg1_ref, recip_ref, z_ref, wl_ref, *out_refs):
        agg = jnp.concatenate([agg0_ref[...], agg1_ref[...]], axis=1)
        mean = agg * recip_ref[...]
        out = (
            jnp.dot(mean, wl_ref[...], preferred_element_type=jnp.float32)
            + z_ref[...]
        )
        if relu:
            out = jnp.maximum(out, 0.0)
        if split_out:
            out_refs[0][...] = out[:, :H]
            out_refs[1][...] = out[:, H:]
        else:
            out_refs[0][...] = out

    if split_out:
        out_specs = [
            pl.BlockSpec((RB, H), lambda i: (i, 0)),
            pl.BlockSpec((RB, H), lambda i: (i, 0)),
        ]
        out_shape = [
            jax.ShapeDtypeStruct((N, H), jnp.float32),
            jax.ShapeDtypeStruct((N, H), jnp.float32),
        ]
    else:
        out_specs = pl.BlockSpec((RB, D), lambda i: (i, 0))
        out_shape = jax.ShapeDtypeStruct((N, D), jnp.float32)

    return pl.pallas_call(
        body,
        grid=(N // RB,),
        in_specs=[
            pl.BlockSpec((RB, H), lambda i: (i, 0)),
            pl.BlockSpec((RB, H), lambda i: (i, 0)),
            pl.BlockSpec((RB, 1), lambda i: (i, 0)),
            pl.BlockSpec((RB, D), lambda i: (i, 0)),
            pl.BlockSpec((D, D), lambda i: (0, 0)),
        ],
        out_specs=out_specs,
        out_shape=out_shape,
    )(agg0, agg1, recip, z, W_l)


@jax.jit
def kernel(x, edge_index, W1_l, b1_l, W1_r, W2_l, b2_l, W2_r):
    src = edge_index[0].astype(jnp.int32)
    dst = edge_index[1].astype(jnp.int32)
    src3d = src.reshape(NS, CHUNKS, K)
    dst3d = dst.reshape(NS, CHUNKS, K)
    x0 = x[:, :H]
    x1 = x[:, H:]

    z1 = _tc_right(x0, x1, W1_r, b1_l.reshape(1, D))
    agg1_0, agg1_1, deg_parts = _sc_segment_sum(
        x0, x1, src3d, dst3d, with_deg=True)
    recip = _tc_recip_deg(deg_parts.reshape(NW, NPAD))
    h0, h1 = _tc_left(agg1_0, agg1_1, recip, z1, W1_l, relu=True,
                      split_out=True)
    z2 = _tc_right(h0, h1, W2_r, b2_l.reshape(1, D))
    agg2_0, agg2_1 = _sc_segment_sum(h0, h1, src3d, dst3d, with_deg=False)
    return _tc_left(agg2_0, agg2_1, recip, z2, W2_l, relu=False,
                    split_out=False)


# speedup vs baseline: 1.1327x; 1.0142x over previous
import functools

import jax
import jax.numpy as jnp
from jax import lax
from jax.experimental import pallas as pl
from jax.experimental.pallas import tpu as pltpu
from jax.experimental.pallas import tpu_sc as plsc

NC = 2
NS = 16
NW = NC * NS
N = 10000
NPAD = 10240
D = 128
H = D // NC
E = 320000
EP = E // NS
K = 80
CHUNKS = EP // K
SUPC = 250
SUP = CHUNKS // SUPC
NBUF = 5
UNROLL = 10
INNER = SUPC // UNROLL
SLAG = 1
GLEAD = NBUF - SLAG
ROWS_PER_TILE = NPAD // NS


def _sc_segment_sum(x0, x1, src3d, dst3d, *, with_deg):
    mesh = plsc.VectorSubcoreMesh(core_axis_name="c", subcore_axis_name="s")

    out_type = [
        jax.ShapeDtypeStruct((NPAD, H), jnp.float32),
        jax.ShapeDtypeStruct((NPAD, H), jnp.float32),
    ]
    if with_deg:
        out_type.append(jax.ShapeDtypeStruct((NW * NPAD,), jnp.float32))

    scratch = [
        pltpu.VMEM_SHARED((NPAD, H), jnp.float32),
        pltpu.VMEM((SUPC, K), jnp.int32),
        pltpu.VMEM((SUPC, K), jnp.int32),
    ] + [pltpu.VMEM((K, H), jnp.float32) for _ in range(NBUF)] + [
        pltpu.VMEM((NPAD,), jnp.float32),
    ] + [pltpu.SemaphoreType.DMA for _ in range(2 * NBUF)]

    def body(x0_hbm, x1_hbm, src_hbm, dst_hbm, agg0_hbm, agg1_hbm, *rest):
        if with_deg:
            degp_hbm = rest[0]
            scr = rest[1:]
        else:
            scr = rest
        agg_sp = scr[0]
        src_v = scr[1]
        dst_v = scr[2]
        bufs = scr[3:3 + NBUF]
        deg_l = scr[3 + NBUF]
        gsem = scr[4 + NBUF:4 + 2 * NBUF]
        ssem = scr[4 + 2 * NBUF:4 + 3 * NBUF]
        rows_v = bufs[0]

        c = lax.axis_index("c")
        s = lax.axis_index("s")
        w = s * NC + c
        zeros16 = jnp.zeros((16,), jnp.float32)
        ones16 = jnp.ones((16,), jnp.float32)

        def zero_rb(i, carry):
            r = i // (H // 16)
            q = lax.rem(i, H // 16)
            rows_v[r, pl.ds(q * 16, 16)] = zeros16
            return carry
        lax.fori_loop(0, K * (H // 16), zero_rb, 0)
        for b in range(ROWS_PER_TILE // K):
            pltpu.sync_copy(rows_v, agg_sp.at[pl.ds(s * ROWS_PER_TILE + b * K, K)])

        if with_deg:
            def zero_deg(i, carry):
                deg_l[pl.ds(i * 16, 16)] = zeros16
                return carry
            lax.fori_loop(0, NPAD // 16, zero_deg, 0)

        plsc.subcore_barrier()

        def main_loop(xh):
            def super_chunk(g, carry):
                pltpu.sync_copy(src_hbm.at[s, pl.ds(g * SUPC, SUPC)], src_v)
                pltpu.sync_copy(dst_hbm.at[s, pl.ds(g * SUPC, SUPC)], dst_v)
                for r0 in range(GLEAD):
                    pltpu.async_copy(xh.at[src_v.at[r0]], bufs[r0], gsem[r0])

                def inner(p, carry2):
                    for b10 in range(UNROLL):
                        r = p * UNROLL + b10
                        B = b10 % NBUF
                        BF = (b10 + NBUF - SLAG) % NBUF
                        pltpu.make_async_copy(
                            xh.at[src_v.at[r]], bufs[B], gsem[B]).wait()
                        pltpu.async_copy(
                            bufs[B], agg_sp.at[dst_v.at[r]], ssem[B], add=True)
                        if b10 < SLAG:
                            @pl.when(p > 0)
                            def _():
                                pltpu.make_async_copy(
                                    bufs[BF], agg_sp.at[dst_v.at[r - SLAG]],
                                    ssem[BF]).wait()
                        else:
                            pltpu.make_async_copy(
                                bufs[BF], agg_sp.at[dst_v.at[r - SLAG]],
                                ssem[BF]).wait()
                        if b10 < UNROLL - GLEAD:
                            pltpu.async_copy(
                                xh.at[src_v.at[r + GLEAD]], bufs[BF],
                                gsem[BF])
                        else:
                            @pl.when(p < INNER - 1)
                            def _():
                                pltpu.async_copy(
                                    xh.at[src_v.at[r + GLEAD]], bufs[BF],
                                    gsem[BF])
                        if with_deg:
                            @pl.when(r // (SUPC // NC) == c)
                            def _():
                                for q in range(K // 16):
                                    idx = dst_v[r, pl.ds(q * 16, 16)]
                                    plsc.addupdate_scatter(deg_l, [idx],
                                                           ones16)
                    return carry2
                lax.fori_loop(0, INNER, inner, 0)
                for t in range(SLAG):
                    rt = SUPC - SLAG + t
                    BT = rt % NBUF
                    pltpu.make_async_copy(
                        bufs[BT], agg_sp.at[dst_v.at[rt]], ssem[BT]).wait()
                return carry
            lax.fori_loop(0, SUP, super_chunk, 0)

        @pl.when(c == 0)
        def _():
            main_loop(x0_hbm)
        @pl.when(c == 1)
        def _():
            main_loop(x1_hbm)

        if with_deg:
            pltpu.sync_copy(deg_l, degp_hbm.at[pl.ds(w * NPAD, NPAD)])

        plsc.subcore_barrier()

        for ci, agg_hbm in ((0, agg0_hbm), (1, agg1_hbm)):
            @pl.when(c == ci)
            def _(agg_hbm=agg_hbm):
                pltpu.sync_copy(
                    agg_sp.at[pl.ds(s * ROWS_PER_TILE, ROWS_PER_TILE)],
                    agg_hbm.at[pl.ds(s * ROWS_PER_TILE, ROWS_PER_TILE)],
                )

    f = pl.kernel(body, out_type=tuple(out_type), mesh=mesh,
                  scratch_types=scratch,
                  compiler_params=pltpu.CompilerParams(
                      needs_layout_passes=False,
                      use_tc_tiling_on_sc=False))
    return f(x0, x1, src3d, dst3d)


def _tc_recip_deg(deg_parts):
    def body(degp_ref, recip_ref):
        deg = jnp.sum(degp_ref[...], axis=0)
        recip_ref[...] = (1.0 / jnp.maximum(deg, 1.0))[:, None]

    return pl.pallas_call(
        body,
        out_shape=jax.ShapeDtypeStruct((NPAD, 1), jnp.float32),
    )(deg_parts)


def _tc_right(x0, x1, W_r, b):
    RB = 1000

    def body(x0_ref, x1_ref, wr_ref, b_ref, z_ref):
        xx = jnp.concatenate([x0_ref[...], x1_ref[...]], axis=1)
        z_ref[...] = (
            jnp.dot(xx, wr_ref[...], preferred_element_type=jnp.float32)
            + b_ref[...]
        )

    return pl.pallas_call(
        body,
        grid=(N // RB,),
        in_specs=[
            pl.BlockSpec((RB, H), lambda i: (i, 0)),
            pl.BlockSpec((RB, H), lambda i: (i, 0)),
            pl.BlockSpec((D, D), lambda i: (0, 0)),
            pl.BlockSpec((1, D), lambda i: (0, 0)),
        ],
        out_specs=pl.BlockSpec((RB, D), lambda i: (i, 0)),
        out_shape=jax.ShapeDtypeStruct((N, D), jnp.float32),
    )(x0, x1, W_r, b)


def _tc_left(agg0, agg1, recip, z, W_l, relu, split_out):
    RB = 1000

    def body(agg0_ref, agg1_ref, recip_ref, z_ref, wl_ref, *out_refs):
        agg = jnp.concatenate([agg0_ref[...], agg1_ref[...]], axis=1)
        mean = agg * recip_ref[...]
        out = (
            jnp.dot(mean, wl_ref[...], preferred_element_type=jnp.float32)
            + z_ref[...]
        )
        if relu:
            out = jnp.maximum(out, 0.0)
        if split_out:
            out_refs[0][...] = out[:, :H]
            out_refs[1][...] = out[:, H:]
        else:
            out_refs[0][...] = out

    if split_out:
        out_specs = [
            pl.BlockSpec((RB, H), lambda i: (i, 0)),
            pl.BlockSpec((RB, H), lambda i: (i, 0)),
        ]
        out_shape = [
            jax.ShapeDtypeStruct((N, H), jnp.float32),
            jax.ShapeDtypeStruct((N, H), jnp.float32),
        ]
    else:
        out_specs = pl.BlockSpec((RB, D), lambda i: (i, 0))
        out_shape = jax.ShapeDtypeStruct((N, D), jnp.float32)

    return pl.pallas_call(
        body,
        grid=(N // RB,),
        in_specs=[
            pl.BlockSpec((RB, H), lambda i: (i, 0)),
            pl.BlockSpec((RB, H), lambda i: (i, 0)),
            pl.BlockSpec((RB, 1), lambda i: (i, 0)),
            pl.BlockSpec((RB, D), lambda i: (i, 0)),
            pl.BlockSpec((D, D), lambda i: (0, 0)),
        ],
        out_specs=out_specs,
        out_shape=out_shape,
    )(agg0, agg1, recip, z, W_l)


@jax.jit
def kernel(x, edge_index, W1_l, b1_l, W1_r, W2_l, b2_l, W2_r):
    src = edge_index[0].astype(jnp.int32)
    dst = edge_index[1].astype(jnp.int32)
    src3d = src.reshape(NS, CHUNKS, K)
    dst3d = dst.reshape(NS, CHUNKS, K)
    x0 = x[:, :H]
    x1 = x[:, H:]

    z1 = _tc_right(x0, x1, W1_r, b1_l.reshape(1, D))
    agg1_0, agg1_1, deg_parts = _sc_segment_sum(
        x0, x1, src3d, dst3d, with_deg=True)
    recip = _tc_recip_deg(deg_parts.reshape(NW, NPAD))
    h0, h1 = _tc_left(agg1_0, agg1_1, recip, z1, W1_l, relu=True,
                      split_out=True)
    z2 = _tc_right(h0, h1, W2_r, b2_l.reshape(1, D))
    agg2_0, agg2_1 = _sc_segment_sum(h0, h1, src3d, dst3d, with_deg=False)
    return _tc_left(agg2_0, agg2_1, recip, z2, W2_l, relu=False,
                    split_out=False)
